# baseline (speedup 1.0000x reference)
import functools

import jax
import jax.numpy as jnp
from jax import lax
from jax.experimental import pallas as pl
from jax.experimental.pallas import tpu as pltpu
from jax.experimental.pallas import tpu_sc as plsc

N = 10000
H = 128
N_PAD = 10112
ROWS_PER_TILE = N_PAD // 16
_G = 8

_MESH = functools.partial(
    plsc.VectorSubcoreMesh, core_axis_name="c", subcore_axis_name="s"
)


def _make_hist_kernel(n_chunks):
    @functools.partial(
        pl.kernel,
        out_type=jax.ShapeDtypeStruct((2, N_PAD, H), jnp.float32),
        mesh=_MESH(),
        scratch_types=[
            pltpu.VMEM((n_chunks, 128), jnp.int32),
            pltpu.VMEM((128, H), jnp.float32),
            pltpu.VMEM_SHARED((N_PAD, H), jnp.float32),
        ],
    )
    def hist_kernel(dst_hbm, zeros_hbm, ones_hbm, out_hbm, idx_v, ones_v, hist_sp):
        c = lax.axis_index("c")
        s = lax.axis_index("s")
        w = c * 16 + s
        base = s * ROWS_PER_TILE
        pltpu.sync_copy(
            zeros_hbm.at[pl.ds(base, ROWS_PER_TILE)],
            hist_sp.at[pl.ds(base, ROWS_PER_TILE)],
        )
        pltpu.sync_copy(dst_hbm.at[w], idx_v)
        pltpu.sync_copy(ones_hbm, ones_v)
        plsc.subcore_barrier()

        def body(i, carry):
            pltpu.sync_copy(ones_v, hist_sp.at[idx_v.at[i]], add=True)
            return carry

        lax.fori_loop(0, n_chunks, body, 0)
        plsc.subcore_barrier()
        pltpu.sync_copy(
            hist_sp.at[pl.ds(base, ROWS_PER_TILE)],
            out_hbm.at[c, pl.ds(base, ROWS_PER_TILE)],
        )

    return hist_kernel


def _make_scatter_kernel(n_chunks):
    assert n_chunks % _G == 0
    n_win = n_chunks // _G
    assert n_win >= 2

    @functools.partial(
        pl.kernel,
        out_type=jax.ShapeDtypeStruct((2, N_PAD, H), jnp.float32),
        mesh=_MESH(),
        scratch_types=[
            pltpu.VMEM((_G, 2, 128), jnp.int32),
            pltpu.VMEM((128, H), jnp.float32),
            pltpu.VMEM((128, H), jnp.float32),
            pltpu.VMEM_SHARED((N_PAD, H), jnp.float32),
            pltpu.SemaphoreType.DMA,
            pltpu.SemaphoreType.DMA,
        ],
    )
    def scatter_kernel(
        g_hbm, sd_hbm, zeros_hbm, out_hbm, slab, rows0, rows1, acc_sp, sem0, sem1
    ):
        c = lax.axis_index("c")
        s = lax.axis_index("s")
        w = c * 16 + s
        base = s * ROWS_PER_TILE
        pltpu.sync_copy(
            zeros_hbm.at[pl.ds(base, ROWS_PER_TILE)],
            acc_sp.at[pl.ds(base, ROWS_PER_TILE)],
        )
        plsc.subcore_barrier()
        bufs = [(rows0, sem0), (rows1, sem1)]

        def win(wi, carry):
            pltpu.sync_copy(sd_hbm.at[w, wi], slab)
            handles = [
                pltpu.async_copy(g_hbm.at[slab.at[b, 0]], bufs[b][0], bufs[b][1])
                for b in range(2)
            ]
            for k in range(_G):
                b = k % 2
                rv, sem = bufs[b]
                handles[b].wait()
                pltpu.sync_copy(rv, acc_sp.at[slab.at[k, 1]], add=True)
                if k + 2 < _G:
                    handles[b] = pltpu.async_copy(
                        g_hbm.at[slab.at[k + 2, 0]], rv, sem
                    )
            return carry

        lax.fori_loop(0, n_win, win, 0)
        plsc.subcore_barrier()
        pltpu.sync_copy(
            acc_sp.at[pl.ds(base, ROWS_PER_TILE)],
            out_hbm.at[c, pl.ds(base, ROWS_PER_TILE)],
        )

    return scatter_kernel


_R = 2528
_GRID = N_PAD // _R


def _dis_vector(h0, h1):
    return lax.rsqrt(h0[:, 0] + h1[:, 0] + 1.0)


def _tc_pre_body(h0_ref, h1_ref, z_ref, w_ref, g_ref):
    dis = _dis_vector(h0_ref[...], h1_ref[...])
    h = jnp.dot(z_ref[...], w_ref[...], preferred_element_type=jnp.float32)
    g_ref[...] = h * dis[:, None]


def _tc_mid_body(h0_ref, h1_ref, a0_ref, a1_ref, g_ref, w_ref, b_ref, gout_ref):
    dis = _dis_vector(h0_ref[...], h1_ref[...])
    x = dis[:, None] * (a0_ref[...] + a1_ref[...] + g_ref[...]) + b_ref[...]
    x = jnp.maximum(x, 0.0)
    h = jnp.dot(x, w_ref[...], preferred_element_type=jnp.float32)
    gout_ref[...] = h * dis[:, None]


def _tc_post_body(h0_ref, h1_ref, a0_ref, a1_ref, g_ref, b_ref, o_ref):
    dis = _dis_vector(h0_ref[...], h1_ref[...])
    x = dis[:, None] * (a0_ref[...] + a1_ref[...] + g_ref[...]) + b_ref[...]
    o_ref[...] = jnp.tanh(x)


_row_spec = pl.BlockSpec((_R, H), lambda i: (i, 0))
_w_spec = pl.BlockSpec((H, H), lambda i: (0, 0))
_b_spec = pl.BlockSpec((1, H), lambda i: (0, 0))

_g_shape = jax.ShapeDtypeStruct((N_PAD, H), jnp.float32)

_tc_pre = pl.pallas_call(
    _tc_pre_body,
    grid=(_GRID,),
    in_specs=[_row_spec, _row_spec, _row_spec, _w_spec],
    out_specs=_row_spec,
    out_shape=_g_shape,
)

_tc_mid = pl.pallas_call(
    _tc_mid_body,
    grid=(_GRID,),
    in_specs=[
        _row_spec,
        _row_spec,
        _row_spec,
        _row_spec,
        _row_spec,
        _w_spec,
        _b_spec,
    ],
    out_specs=_row_spec,
    out_shape=_g_shape,
)

_tc_post = pl.pallas_call(
    _tc_post_body,
    grid=(_GRID,),
    in_specs=[
        _row_spec,
        _row_spec,
        _row_spec,
        _row_spec,
        _row_spec,
        _b_spec,
    ],
    out_specs=_row_spec,
    out_shape=_g_shape,
)


def kernel(z, edge_index, W1, b1, W2, b2):
    E = edge_index.shape[1]
    q = 32 * 128 * _G
    e_pad = -(-E // q) * q
    n_chunks = e_pad // (32 * 128)
    n_win = n_chunks // _G

    pad = jnp.full((e_pad - E,), N, dtype=jnp.int32)
    src = jnp.concatenate([edge_index[0], pad])
    dst = jnp.concatenate([edge_index[1], pad])
    sd = jnp.stack(
        [src.reshape(32, n_win, _G, 128), dst.reshape(32, n_win, _G, 128)],
        axis=3,
    )
    dst3 = dst.reshape(32, n_chunks, 128)

    zeros_h = jnp.zeros((N_PAD, H), jnp.float32)
    ones_tab = jnp.ones((128, H), jnp.float32)
    z_pad = jnp.concatenate([z, jnp.zeros((N_PAD - N, H), z.dtype)])
    b1r = b1.reshape(1, H)
    b2r = b2.reshape(1, H)

    scatter = _make_scatter_kernel(n_chunks)

    hist = _make_hist_kernel(n_chunks)(dst3, zeros_h, ones_tab)
    h0, h1 = hist[0], hist[1]

    g1 = _tc_pre(h0, h1, z_pad, W1)
    acc1 = scatter(g1, sd, zeros_h)
    g2 = _tc_mid(h0, h1, acc1[0], acc1[1], g1, W2, b1r)
    acc2 = scatter(g2, sd, zeros_h)
    out = _tc_post(h0, h1, acc2[0], acc2[1], g2, b2r)
    return out[:N]


# speedup vs baseline: 1.3480x; 1.3480x over previous
import functools

import jax
import jax.numpy as jnp
from jax import lax
from jax.experimental import pallas as pl
from jax.experimental.pallas import tpu as pltpu
from jax.experimental.pallas import tpu_sc as plsc

N = 10000
H = 128
N_PAD = 10112
ROWS_PER_TILE = N_PAD // 16

_MESH = functools.partial(
    plsc.VectorSubcoreMesh, core_axis_name="c", subcore_axis_name="s"
)


def _make_hist_kernel(n_chunks):
    @functools.partial(
        pl.kernel,
        out_type=jax.ShapeDtypeStruct((2, N_PAD, H), jnp.float32),
        mesh=_MESH(),
        scratch_types=[
            pltpu.VMEM((n_chunks, 128), jnp.int32),
            pltpu.VMEM((128, H), jnp.float32),
            pltpu.VMEM_SHARED((N_PAD, H), jnp.float32),
        ],
    )
    def hist_kernel(dst_hbm, zeros_hbm, ones_hbm, out_hbm, idx_v, ones_v, hist_sp):
        c = lax.axis_index("c")
        s = lax.axis_index("s")
        w = c * 16 + s
        base = s * ROWS_PER_TILE
        pltpu.sync_copy(
            zeros_hbm.at[pl.ds(base, ROWS_PER_TILE)],
            hist_sp.at[pl.ds(base, ROWS_PER_TILE)],
        )
        pltpu.sync_copy(dst_hbm.at[w], idx_v)
        pltpu.sync_copy(ones_hbm, ones_v)
        plsc.subcore_barrier()

        def body(i, carry):
            pltpu.sync_copy(ones_v, hist_sp.at[idx_v.at[i]], add=True)
            return carry

        lax.fori_loop(0, n_chunks, body, 0)
        plsc.subcore_barrier()
        pltpu.sync_copy(
            hist_sp.at[pl.ds(base, ROWS_PER_TILE)],
            out_hbm.at[c, pl.ds(base, ROWS_PER_TILE)],
        )

    return hist_kernel


def _make_scatter_kernel(n_chunks):
    @functools.partial(
        pl.kernel,
        out_type=jax.ShapeDtypeStruct((2, N_PAD, H), jnp.float32),
        mesh=_MESH(),
        scratch_types=[
            pltpu.VMEM((n_chunks, 128), jnp.int32),
            pltpu.VMEM((n_chunks, 128), jnp.int32),
            pltpu.VMEM((128, H), jnp.float32),
            pltpu.VMEM_SHARED((N_PAD, H), jnp.float32),
            pltpu.SemaphoreType.DMA,
        ],
    )
    def scatter_kernel(
        g_hbm, src_hbm, dst_hbm, zeros_hbm, out_hbm, idx_s, idx_d, rows_v, acc_sp, sem
    ):
        c = lax.axis_index("c")
        s = lax.axis_index("s")
        w = c * 16 + s
        base = s * ROWS_PER_TILE
        pltpu.sync_copy(
            zeros_hbm.at[pl.ds(base, ROWS_PER_TILE)],
            acc_sp.at[pl.ds(base, ROWS_PER_TILE)],
        )
        pltpu.sync_copy(src_hbm.at[w], idx_s)
        pltpu.sync_copy(dst_hbm.at[w], idx_d)
        plsc.subcore_barrier()

        def body(i, carry):
            pltpu.async_copy(g_hbm.at[idx_s.at[i]], rows_v, sem).wait()
            pltpu.sync_copy(rows_v, acc_sp.at[idx_d.at[i]], add=True)
            return carry

        lax.fori_loop(0, n_chunks, body, 0)
        plsc.subcore_barrier()
        pltpu.sync_copy(
            acc_sp.at[pl.ds(base, ROWS_PER_TILE)],
            out_hbm.at[c, pl.ds(base, ROWS_PER_TILE)],
        )

    return scatter_kernel


_R = 2528
_GRID = N_PAD // _R


def _dis_vector(h0, h1):
    return lax.rsqrt(h0[:, 0] + h1[:, 0] + 1.0)


def _tc_pre_body(h0_ref, h1_ref, z_ref, w_ref, g_ref):
    dis = _dis_vector(h0_ref[...], h1_ref[...])
    h = jnp.dot(z_ref[...], w_ref[...], preferred_element_type=jnp.float32)
    g_ref[...] = h * dis[:, None]


def _tc_mid_body(h0_ref, h1_ref, a0_ref, a1_ref, g_ref, w_ref, b_ref, gout_ref):
    dis = _dis_vector(h0_ref[...], h1_ref[...])
    x = dis[:, None] * (a0_ref[...] + a1_ref[...] + g_ref[...]) + b_ref[...]
    x = jnp.maximum(x, 0.0)
    h = jnp.dot(x, w_ref[...], preferred_element_type=jnp.float32)
    gout_ref[...] = h * dis[:, None]


def _tc_post_body(h0_ref, h1_ref, a0_ref, a1_ref, g_ref, b_ref, o_ref):
    dis = _dis_vector(h0_ref[...], h1_ref[...])
    x = dis[:, None] * (a0_ref[...] + a1_ref[...] + g_ref[...]) + b_ref[...]
    o_ref[...] = jnp.tanh(x)


_row_spec = pl.BlockSpec((_R, H), lambda i: (i, 0))
_w_spec = pl.BlockSpec((H, H), lambda i: (0, 0))
_b_spec = pl.BlockSpec((1, H), lambda i: (0, 0))

_g_shape = jax.ShapeDtypeStruct((N_PAD, H), jnp.float32)

_tc_pre = pl.pallas_call(
    _tc_pre_body,
    grid=(_GRID,),
    in_specs=[_row_spec, _row_spec, _row_spec, _w_spec],
    out_specs=_row_spec,
    out_shape=_g_shape,
)

_tc_mid = pl.pallas_call(
    _tc_mid_body,
    grid=(_GRID,),
    in_specs=[
        _row_spec,
        _row_spec,
        _row_spec,
        _row_spec,
        _row_spec,
        _w_spec,
        _b_spec,
    ],
    out_specs=_row_spec,
    out_shape=_g_shape,
)

_tc_post = pl.pallas_call(
    _tc_post_body,
    grid=(_GRID,),
    in_specs=[
        _row_spec,
        _row_spec,
        _row_spec,
        _row_spec,
        _row_spec,
        _b_spec,
    ],
    out_specs=_row_spec,
    out_shape=_g_shape,
)


def kernel(z, edge_index, W1, b1, W2, b2):
    E = edge_index.shape[1]
    q = 32 * 128
    e_pad = -(-E // q) * q
    n_chunks = e_pad // q

    pad = jnp.full((e_pad - E,), N, dtype=jnp.int32)
    src3 = jnp.concatenate([edge_index[0], pad]).reshape(32, n_chunks, 128)
    dst3 = jnp.concatenate([edge_index[1], pad]).reshape(32, n_chunks, 128)

    zeros_h = jnp.zeros((N_PAD, H), jnp.float32)
    ones_tab = jnp.ones((128, H), jnp.float32)
    z_pad = jnp.concatenate([z, jnp.zeros((N_PAD - N, H), z.dtype)])
    b1r = b1.reshape(1, H)
    b2r = b2.reshape(1, H)

    scatter = _make_scatter_kernel(n_chunks)

    hist = _make_hist_kernel(n_chunks)(dst3, zeros_h, ones_tab)
    h0, h1 = hist[0], hist[1]

    g1 = _tc_pre(h0, h1, z_pad, W1)
    acc1 = scatter(g1, src3, dst3, zeros_h)
    g2 = _tc_mid(h0, h1, acc1[0], acc1[1], g1, W2, b1r)
    acc2 = scatter(g2, src3, dst3, zeros_h)
    out = _tc_post(h0, h1, acc2[0], acc2[1], g2, b2r)
    return out[:N]
